# Initial kernel scaffold; baseline (speedup 1.0000x reference)
import jax
import jax.numpy as jnp
from jax.experimental import pallas as pl


def kernel(x, edge_index, W1, b1, W2, b2, W3, b3, Wl, bl):
    raise NotImplementedError("write your pallas kernel here")


# speedup vs baseline: 29.0199x; 29.0199x over previous
import functools

import jax
import jax.numpy as jnp
from jax import lax
from jax.experimental import pallas as pl
from jax.experimental.pallas import tpu as pltpu
from jax.experimental.pallas import tpu_sc as plsc

N = 10000
D_IN = 128
DIM = 16
N_CLASSES = 16
E = 320000

NC = 2
NS = 16
NT = NC * NS
CH = 128
CHUNKS = 79
E_PAD = NT * CHUNKS * CH
SCRAP = 512
NP = N + SCRAP
RPT = 632
RPT_LAST = N - (NS - 1) * RPT


def _over_rows(s, fn):
  @pl.when(s < NS - 1)
  def _():
    fn(pl.ds(s * RPT, RPT))

  @pl.when(s == NS - 1)
  def _():
    fn(pl.ds((NS - 1) * RPT, RPT_LAST))

@functools.lru_cache(maxsize=None)
def _make_sc_agg(with_gather):
  mesh = plsc.VectorSubcoreMesh(core_axis_name="c", subcore_axis_name="s",
                                num_cores=NC, num_subcores=NS)

  scratch = [
      pltpu.VMEM((CHUNKS, CH), jnp.int32),
      pltpu.VMEM((CHUNKS, CH), jnp.int32),
      pltpu.VMEM((CH, DIM), jnp.float32),
      pltpu.VMEM_SHARED((NP, DIM), jnp.float32),
      pltpu.SemaphoreType.DMA,
  ]

  def body(y_hbm, src_hbm, dst_hbm, zeros_hbm, ones_hbm, out_hbm,
           src_v, dst_v, rows_v, z_sh, sem):
    c = lax.axis_index("c")
    s = lax.axis_index("s")
    wid = s * NC + c

    if with_gather:
      @pl.when(c == 0)
      def _():
        _over_rows(s, lambda r: pltpu.sync_copy(y_hbm.at[r], z_sh.at[r]))

      @pl.when(c != 0)
      def _():
        _over_rows(s, lambda r: pltpu.sync_copy(zeros_hbm.at[r], z_sh.at[r]))
    else:
      _over_rows(s, lambda r: pltpu.sync_copy(zeros_hbm.at[r], z_sh.at[r]))
      pltpu.sync_copy(ones_hbm, rows_v)

    pltpu.sync_copy(src_hbm.at[wid], src_v)
    pltpu.sync_copy(dst_hbm.at[wid], dst_v)
    plsc.subcore_barrier()

    def step(j, _):
      if with_gather:
        pltpu.async_copy(y_hbm.at[src_v.at[j]], rows_v, sem).wait()
      pltpu.sync_copy(rows_v, z_sh.at[dst_v.at[j]], add=True)
      return 0

    lax.fori_loop(0, CHUNKS, step, 0)
    plsc.subcore_barrier()
    _over_rows(s, lambda r: pltpu.sync_copy(z_sh.at[r], out_hbm.at[c, r]))

  return pl.kernel(
      body,
      out_type=jax.ShapeDtypeStruct((NC, N, DIM), jnp.float32),
      mesh=mesh,
      scratch_types=scratch,
      compiler_params=pltpu.CompilerParams(use_tc_tiling_on_sc=False),
  )


BT = 2000
GRID = N // BT


def _tc_prep1_body(d_ref, x_ref, w_ref, y_ref, dinv_ref):
  deg = d_ref[0] + d_ref[1] + 1.0
  dinv = lax.rsqrt(deg)
  dinv_ref[...] = dinv
  y_ref[...] = dinv * jnp.dot(x_ref[...], w_ref[...],
                              preferred_element_type=jnp.float32)


_tc_prep1 = pl.pallas_call(
    _tc_prep1_body,
    grid=(GRID,),
    in_specs=[
        pl.BlockSpec((NC, BT, DIM), lambda i: (0, i, 0)),
        pl.BlockSpec((BT, D_IN), lambda i: (i, 0)),
        pl.BlockSpec((D_IN, DIM), lambda i: (0, 0)),
    ],
    out_specs=[
        pl.BlockSpec((BT, DIM), lambda i: (i, 0)),
        pl.BlockSpec((BT, DIM), lambda i: (i, 0)),
    ],
    out_shape=[
        jax.ShapeDtypeStruct((N, DIM), jnp.float32),
        jax.ShapeDtypeStruct((N, DIM), jnp.float32),
    ],
)


def _tc_mid_body(z_ref, dinv_ref, b_ref, w_ref, y_ref):
  dinv = dinv_ref[...]
  h = jax.nn.relu(dinv * (z_ref[0] + z_ref[1]) + b_ref[...])
  y_ref[...] = dinv * jnp.dot(h, w_ref[...],
                              preferred_element_type=jnp.float32)


_tc_mid = pl.pallas_call(
    _tc_mid_body,
    grid=(GRID,),
    in_specs=[
        pl.BlockSpec((NC, BT, DIM), lambda i: (0, i, 0)),
        pl.BlockSpec((BT, DIM), lambda i: (i, 0)),
        pl.BlockSpec((1, DIM), lambda i: (0, 0)),
        pl.BlockSpec((DIM, DIM), lambda i: (0, 0)),
    ],
    out_specs=pl.BlockSpec((BT, DIM), lambda i: (i, 0)),
    out_shape=jax.ShapeDtypeStruct((N, DIM), jnp.float32),
)


def _tc_final_body(z_ref, dinv_ref, b_ref, wl_ref, bl_ref, o_ref):
  h = dinv_ref[...] * (z_ref[0] + z_ref[1]) + b_ref[...]
  lg = jnp.dot(h, wl_ref[...], preferred_element_type=jnp.float32)
  lg = lg + bl_ref[...]
  m = jnp.max(lg, axis=1, keepdims=True)
  lse = m + jnp.log(jnp.sum(jnp.exp(lg - m), axis=1, keepdims=True))
  o_ref[...] = lg - lse


_tc_final = pl.pallas_call(
    _tc_final_body,
    grid=(GRID,),
    in_specs=[
        pl.BlockSpec((NC, BT, DIM), lambda i: (0, i, 0)),
        pl.BlockSpec((BT, DIM), lambda i: (i, 0)),
        pl.BlockSpec((1, DIM), lambda i: (0, 0)),
        pl.BlockSpec((DIM, N_CLASSES), lambda i: (0, 0)),
        pl.BlockSpec((1, N_CLASSES), lambda i: (0, 0)),
    ],
    out_specs=pl.BlockSpec((BT, N_CLASSES), lambda i: (i, 0)),
    out_shape=jax.ShapeDtypeStruct((N, N_CLASSES), jnp.float32),
)


def kernel(x, edge_index, W1, b1, W2, b2, W3, b3, Wl, bl):
  src = edge_index[0]
  dst = edge_index[1]
  padlen = E_PAD - E
  pad_src = jnp.zeros((padlen,), jnp.int32)
  pad_dst = N + (jnp.arange(padlen, dtype=jnp.int32) % SCRAP)
  src_p = jnp.concatenate([src, pad_src]).reshape(NT, CHUNKS, CH)
  dst_p = jnp.concatenate([dst, pad_dst]).reshape(NT, CHUNKS, CH)

  zeros = jnp.zeros((N, DIM), jnp.float32)
  ones = jnp.ones((CH, DIM), jnp.float32)
  dummy_y = zeros

  sc_layer = _make_sc_agg(True)
  sc_deg = _make_sc_agg(False)

  deg_parts = sc_deg(dummy_y, src_p, dst_p, zeros, ones)
  y1, dinv = _tc_prep1(deg_parts, x, W1)
  z1 = sc_layer(y1, src_p, dst_p, zeros, ones)
  y2 = _tc_mid(z1, dinv, b1.reshape(1, DIM), W2)
  z2 = sc_layer(y2, src_p, dst_p, zeros, ones)
  y3 = _tc_mid(z2, dinv, b2.reshape(1, DIM), W3)
  z3 = sc_layer(y3, src_p, dst_p, zeros, ones)
  return _tc_final(z3, dinv, b3.reshape(1, DIM), Wl,
                   bl.reshape(1, N_CLASSES))


# speedup vs baseline: 34.5436x; 1.1903x over previous
import functools

import jax
import jax.numpy as jnp
from jax import lax
from jax.experimental import pallas as pl
from jax.experimental.pallas import tpu as pltpu
from jax.experimental.pallas import tpu_sc as plsc

N = 10000
D_IN = 128
DIM = 16
N_CLASSES = 16
E = 320000

NC = 2
NS = 16
NT = NC * NS
CH = 128
CHUNKS = 80
NBUF = 4
NOUT = CHUNKS // NBUF
E_PAD = NT * CHUNKS * CH
SCRAP = 512
NP = N + SCRAP
RPT = 632
RPT_LAST = N - (NS - 1) * RPT


def _over_rows(s, fn):
  @pl.when(s < NS - 1)
  def _():
    fn(pl.ds(s * RPT, RPT))

  @pl.when(s == NS - 1)
  def _():
    fn(pl.ds((NS - 1) * RPT, RPT_LAST))

@functools.lru_cache(maxsize=None)
def _make_sc_agg(with_gather):
  mesh = plsc.VectorSubcoreMesh(core_axis_name="c", subcore_axis_name="s",
                                num_cores=NC, num_subcores=NS)

  scratch = [
      pltpu.VMEM((CHUNKS, CH), jnp.int32),
      pltpu.VMEM((CHUNKS, CH), jnp.int32),
      pltpu.VMEM((NBUF, CH, DIM), jnp.float32),
      pltpu.VMEM_SHARED((NP, DIM), jnp.float32),
      pltpu.SemaphoreType.DMA((NBUF,)),
  ]

  def body(y_hbm, src_hbm, dst_hbm, zeros_hbm, ones_hbm, out_hbm,
           src_v, dst_v, rows_v, z_sh, sem):
    c = lax.axis_index("c")
    s = lax.axis_index("s")
    wid = s * NC + c

    if with_gather:
      @pl.when(c == 0)
      def _():
        _over_rows(s, lambda r: pltpu.sync_copy(y_hbm.at[r], z_sh.at[r]))

      @pl.when(c != 0)
      def _():
        _over_rows(s, lambda r: pltpu.sync_copy(zeros_hbm.at[r], z_sh.at[r]))
    else:
      _over_rows(s, lambda r: pltpu.sync_copy(zeros_hbm.at[r], z_sh.at[r]))
      pltpu.sync_copy(ones_hbm, rows_v.at[0])

    pltpu.sync_copy(src_hbm.at[wid], src_v)
    pltpu.sync_copy(dst_hbm.at[wid], dst_v)
    plsc.subcore_barrier()

    if with_gather:
      for b in range(NBUF):
        pltpu.async_copy(y_hbm.at[src_v.at[b]], rows_v.at[b], sem.at[b])

      def outer(o, _):
        for b in range(NBUF):
          j = o * NBUF + b
          pltpu.make_async_copy(
              y_hbm.at[src_v.at[j]], rows_v.at[b], sem.at[b]).wait()
          pltpu.sync_copy(rows_v.at[b], z_sh.at[dst_v.at[j]], add=True)
          nxt = j + NBUF

          @pl.when(nxt < CHUNKS)
          def _():
            pltpu.async_copy(y_hbm.at[src_v.at[nxt]], rows_v.at[b],
                             sem.at[b])
        return 0

      lax.fori_loop(0, NOUT, outer, 0)
    else:

      def step(j, _):
        pltpu.sync_copy(rows_v.at[0], z_sh.at[dst_v.at[j]], add=True)
        return 0

      lax.fori_loop(0, CHUNKS, step, 0)
    plsc.subcore_barrier()
    _over_rows(s, lambda r: pltpu.sync_copy(z_sh.at[r], out_hbm.at[c, r]))

  return pl.kernel(
      body,
      out_type=jax.ShapeDtypeStruct((NC, N, DIM), jnp.float32),
      mesh=mesh,
      scratch_types=scratch,
      compiler_params=pltpu.CompilerParams(use_tc_tiling_on_sc=False),
  )


BT = 2000
GRID = N // BT


def _tc_prep1_body(d_ref, x_ref, w_ref, y_ref, dinv_ref):
  deg = d_ref[0] + d_ref[1] + 1.0
  dinv = lax.rsqrt(deg)
  dinv_ref[...] = dinv
  y_ref[...] = dinv * jnp.dot(x_ref[...], w_ref[...],
                              preferred_element_type=jnp.float32)


_tc_prep1 = pl.pallas_call(
    _tc_prep1_body,
    grid=(GRID,),
    in_specs=[
        pl.BlockSpec((NC, BT, DIM), lambda i: (0, i, 0)),
        pl.BlockSpec((BT, D_IN), lambda i: (i, 0)),
        pl.BlockSpec((D_IN, DIM), lambda i: (0, 0)),
    ],
    out_specs=[
        pl.BlockSpec((BT, DIM), lambda i: (i, 0)),
        pl.BlockSpec((BT, DIM), lambda i: (i, 0)),
    ],
    out_shape=[
        jax.ShapeDtypeStruct((N, DIM), jnp.float32),
        jax.ShapeDtypeStruct((N, DIM), jnp.float32),
    ],
)


def _tc_mid_body(z_ref, dinv_ref, b_ref, w_ref, y_ref):
  dinv = dinv_ref[...]
  h = jax.nn.relu(dinv * (z_ref[0] + z_ref[1]) + b_ref[...])
  y_ref[...] = dinv * jnp.dot(h, w_ref[...],
                              preferred_element_type=jnp.float32)


_tc_mid = pl.pallas_call(
    _tc_mid_body,
    grid=(GRID,),
    in_specs=[
        pl.BlockSpec((NC, BT, DIM), lambda i: (0, i, 0)),
        pl.BlockSpec((BT, DIM), lambda i: (i, 0)),
        pl.BlockSpec((1, DIM), lambda i: (0, 0)),
        pl.BlockSpec((DIM, DIM), lambda i: (0, 0)),
    ],
    out_specs=pl.BlockSpec((BT, DIM), lambda i: (i, 0)),
    out_shape=jax.ShapeDtypeStruct((N, DIM), jnp.float32),
)


def _tc_final_body(z_ref, dinv_ref, b_ref, wl_ref, bl_ref, o_ref):
  h = dinv_ref[...] * (z_ref[0] + z_ref[1]) + b_ref[...]
  lg = jnp.dot(h, wl_ref[...], preferred_element_type=jnp.float32)
  lg = lg + bl_ref[...]
  m = jnp.max(lg, axis=1, keepdims=True)
  lse = m + jnp.log(jnp.sum(jnp.exp(lg - m), axis=1, keepdims=True))
  o_ref[...] = lg - lse


_tc_final = pl.pallas_call(
    _tc_final_body,
    grid=(GRID,),
    in_specs=[
        pl.BlockSpec((NC, BT, DIM), lambda i: (0, i, 0)),
        pl.BlockSpec((BT, DIM), lambda i: (i, 0)),
        pl.BlockSpec((1, DIM), lambda i: (0, 0)),
        pl.BlockSpec((DIM, N_CLASSES), lambda i: (0, 0)),
        pl.BlockSpec((1, N_CLASSES), lambda i: (0, 0)),
    ],
    out_specs=pl.BlockSpec((BT, N_CLASSES), lambda i: (i, 0)),
    out_shape=jax.ShapeDtypeStruct((N, N_CLASSES), jnp.float32),
)


def kernel(x, edge_index, W1, b1, W2, b2, W3, b3, Wl, bl):
  src = edge_index[0]
  dst = edge_index[1]
  padlen = E_PAD - E
  pad_src = jnp.zeros((padlen,), jnp.int32)
  pad_dst = N + (jnp.arange(padlen, dtype=jnp.int32) % SCRAP)
  src_p = jnp.concatenate([src, pad_src]).reshape(NT, CHUNKS, CH)
  dst_p = jnp.concatenate([dst, pad_dst]).reshape(NT, CHUNKS, CH)

  zeros = jnp.zeros((N, DIM), jnp.float32)
  ones = jnp.ones((CH, DIM), jnp.float32)
  dummy_y = zeros

  sc_layer = _make_sc_agg(True)
  sc_deg = _make_sc_agg(False)

  deg_parts = sc_deg(dummy_y, src_p, dst_p, zeros, ones)
  y1, dinv = _tc_prep1(deg_parts, x, W1)
  z1 = sc_layer(y1, src_p, dst_p, zeros, ones)
  y2 = _tc_mid(z1, dinv, b1.reshape(1, DIM), W2)
  z2 = sc_layer(y2, src_p, dst_p, zeros, ones)
  y3 = _tc_mid(z2, dinv, b2.reshape(1, DIM), W3)
  z3 = sc_layer(y3, src_p, dst_p, zeros, ones)
  return _tc_final(z3, dinv, b3.reshape(1, DIM), Wl,
                   bl.reshape(1, N_CLASSES))


# speedup vs baseline: 37.8198x; 1.0948x over previous
import functools

import jax
import jax.numpy as jnp
from jax import lax
from jax.experimental import pallas as pl
from jax.experimental.pallas import tpu as pltpu
from jax.experimental.pallas import tpu_sc as plsc

N = 10000
D_IN = 128
DIM = 16
N_CLASSES = 16
E = 320000

NC = 2
NS = 16
NT = NC * NS
CH = 128
CHUNKS = 80
NBUF = 8
NOUT = CHUNKS // NBUF
E_PAD = NT * CHUNKS * CH
SCRAP = 512
NP = N + SCRAP
RPT = 632
RPT_LAST = N - (NS - 1) * RPT


def _over_rows(s, fn):
  @pl.when(s < NS - 1)
  def _():
    fn(pl.ds(s * RPT, RPT))

  @pl.when(s == NS - 1)
  def _():
    fn(pl.ds((NS - 1) * RPT, RPT_LAST))

@functools.lru_cache(maxsize=None)
def _make_sc_agg(with_gather):
  mesh = plsc.VectorSubcoreMesh(core_axis_name="c", subcore_axis_name="s",
                                num_cores=NC, num_subcores=NS)

  scratch = [
      pltpu.VMEM((CHUNKS, CH), jnp.int32),
      pltpu.VMEM((CHUNKS, CH), jnp.int32),
      pltpu.VMEM((NBUF, CH, DIM), jnp.float32),
      pltpu.VMEM_SHARED((NP, DIM), jnp.float32),
      pltpu.SemaphoreType.DMA((NBUF,)),
  ]

  def body(y_hbm, src_hbm, dst_hbm, zeros_hbm, ones_hbm, out_hbm,
           src_v, dst_v, rows_v, z_sh, sem):
    c = lax.axis_index("c")
    s = lax.axis_index("s")
    wid = s * NC + c

    _over_rows(s, lambda r: pltpu.sync_copy(zeros_hbm.at[r], z_sh.at[r]))
    if not with_gather:
      pltpu.sync_copy(ones_hbm, rows_v.at[0])

    pltpu.sync_copy(src_hbm.at[wid], src_v)
    pltpu.sync_copy(dst_hbm.at[wid], dst_v)
    plsc.subcore_barrier()

    if with_gather:
      for b in range(NBUF):
        pltpu.async_copy(y_hbm.at[src_v.at[b]], rows_v.at[b], sem.at[b])

      def outer(o, _):
        for b in range(NBUF):
          j = o * NBUF + b
          pltpu.make_async_copy(
              y_hbm.at[src_v.at[j]], rows_v.at[b], sem.at[b]).wait()
          pltpu.sync_copy(rows_v.at[b], z_sh.at[dst_v.at[j]], add=True)
          nxt = j + NBUF

          @pl.when(nxt < CHUNKS)
          def _():
            pltpu.async_copy(y_hbm.at[src_v.at[nxt]], rows_v.at[b],
                             sem.at[b])
        return 0

      lax.fori_loop(0, NOUT, outer, 0)
    else:

      def step(j, _):
        pltpu.sync_copy(rows_v.at[0], z_sh.at[dst_v.at[j]], add=True)
        return 0

      lax.fori_loop(0, CHUNKS, step, 0)
    plsc.subcore_barrier()
    _over_rows(s, lambda r: pltpu.sync_copy(z_sh.at[r], out_hbm.at[c, r]))

  return pl.kernel(
      body,
      out_type=jax.ShapeDtypeStruct((NC, N, DIM), jnp.float32),
      mesh=mesh,
      scratch_types=scratch,
      compiler_params=pltpu.CompilerParams(use_tc_tiling_on_sc=False),
  )


BT = 2000
GRID = N // BT


def _tc_prep1_body(d_ref, x_ref, w_ref, y_ref, dinv_ref):
  deg = d_ref[0] + d_ref[1] + 1.0
  dinv = lax.rsqrt(deg)
  dinv_ref[...] = dinv
  y_ref[...] = dinv * jnp.dot(x_ref[...], w_ref[...],
                              preferred_element_type=jnp.float32)


_tc_prep1 = pl.pallas_call(
    _tc_prep1_body,
    grid=(GRID,),
    in_specs=[
        pl.BlockSpec((NC, BT, DIM), lambda i: (0, i, 0)),
        pl.BlockSpec((BT, D_IN), lambda i: (i, 0)),
        pl.BlockSpec((D_IN, DIM), lambda i: (0, 0)),
    ],
    out_specs=[
        pl.BlockSpec((BT, DIM), lambda i: (i, 0)),
        pl.BlockSpec((BT, DIM), lambda i: (i, 0)),
    ],
    out_shape=[
        jax.ShapeDtypeStruct((N, DIM), jnp.float32),
        jax.ShapeDtypeStruct((N, DIM), jnp.float32),
    ],
)


def _tc_mid_body(z_ref, yin_ref, dinv_ref, b_ref, w_ref, y_ref):
  dinv = dinv_ref[...]
  h = jax.nn.relu(dinv * (z_ref[0] + z_ref[1] + yin_ref[...]) + b_ref[...])
  y_ref[...] = dinv * jnp.dot(h, w_ref[...],
                              preferred_element_type=jnp.float32)


_tc_mid = pl.pallas_call(
    _tc_mid_body,
    grid=(GRID,),
    in_specs=[
        pl.BlockSpec((NC, BT, DIM), lambda i: (0, i, 0)),
        pl.BlockSpec((BT, DIM), lambda i: (i, 0)),
        pl.BlockSpec((BT, DIM), lambda i: (i, 0)),
        pl.BlockSpec((1, DIM), lambda i: (0, 0)),
        pl.BlockSpec((DIM, DIM), lambda i: (0, 0)),
    ],
    out_specs=pl.BlockSpec((BT, DIM), lambda i: (i, 0)),
    out_shape=jax.ShapeDtypeStruct((N, DIM), jnp.float32),
)


def _tc_final_body(z_ref, yin_ref, dinv_ref, b_ref, wl_ref, bl_ref, o_ref):
  h = dinv_ref[...] * (z_ref[0] + z_ref[1] + yin_ref[...]) + b_ref[...]
  lg = jnp.dot(h, wl_ref[...], preferred_element_type=jnp.float32)
  lg = lg + bl_ref[...]
  m = jnp.max(lg, axis=1, keepdims=True)
  lse = m + jnp.log(jnp.sum(jnp.exp(lg - m), axis=1, keepdims=True))
  o_ref[...] = lg - lse


_tc_final = pl.pallas_call(
    _tc_final_body,
    grid=(GRID,),
    in_specs=[
        pl.BlockSpec((NC, BT, DIM), lambda i: (0, i, 0)),
        pl.BlockSpec((BT, DIM), lambda i: (i, 0)),
        pl.BlockSpec((BT, DIM), lambda i: (i, 0)),
        pl.BlockSpec((1, DIM), lambda i: (0, 0)),
        pl.BlockSpec((DIM, N_CLASSES), lambda i: (0, 0)),
        pl.BlockSpec((1, N_CLASSES), lambda i: (0, 0)),
    ],
    out_specs=pl.BlockSpec((BT, N_CLASSES), lambda i: (i, 0)),
    out_shape=jax.ShapeDtypeStruct((N, N_CLASSES), jnp.float32),
)


def kernel(x, edge_index, W1, b1, W2, b2, W3, b3, Wl, bl):
  padt = (E_PAD - E) // NT
  src_t = edge_index[0].reshape(NT, E // NT)
  dst_t = edge_index[1].reshape(NT, E // NT)
  pad_src = jnp.zeros((NT, padt), jnp.int32)
  pad_dst = jnp.broadcast_to(
      N + (jnp.arange(padt, dtype=jnp.int32) % SCRAP), (NT, padt))
  src_p = jnp.concatenate([src_t, pad_src], axis=1).reshape(NT, CHUNKS, CH)
  dst_p = jnp.concatenate([dst_t, pad_dst], axis=1).reshape(NT, CHUNKS, CH)

  zeros = jnp.zeros((N, DIM), jnp.float32)
  ones = jnp.ones((CH, DIM), jnp.float32)
  dummy_y = zeros

  sc_layer = _make_sc_agg(True)
  sc_deg = _make_sc_agg(False)

  deg_parts = sc_deg(dummy_y, src_p, dst_p, zeros, ones)
  y1, dinv = _tc_prep1(deg_parts, x, W1)
  z1 = sc_layer(y1, src_p, dst_p, zeros, ones)
  y2 = _tc_mid(z1, y1, dinv, b1.reshape(1, DIM), W2)
  z2 = sc_layer(y2, src_p, dst_p, zeros, ones)
  y3 = _tc_mid(z2, y2, dinv, b2.reshape(1, DIM), W3)
  z3 = sc_layer(y3, src_p, dst_p, zeros, ones)
  return _tc_final(z3, y3, dinv, b3.reshape(1, DIM), Wl,
                   bl.reshape(1, N_CLASSES))


# speedup vs baseline: 52.9176x; 1.3992x over previous
import functools

import jax
import jax.numpy as jnp
from jax import lax
from jax.experimental import pallas as pl
from jax.experimental.pallas import tpu as pltpu
from jax.experimental.pallas import tpu_sc as plsc

N = 10000
D_IN = 128
DIM = 16
N_CLASSES = 16
E = 320000

NC = 2
NS = 16
NT = NC * NS
CH = 128
CHUNKS = 80
NBUF = 8
NOUT = CHUNKS // NBUF
E_PAD = NT * CHUNKS * CH
SCRAP = 512
NP = N + SCRAP
RPT = 632
RPT_LAST = N - (NS - 1) * RPT


def _over_rows(s, fn):
  @pl.when(s < NS - 1)
  def _():
    fn(pl.ds(s * RPT, RPT))

  @pl.when(s == NS - 1)
  def _():
    fn(pl.ds((NS - 1) * RPT, RPT_LAST))

@functools.lru_cache(maxsize=None)
def _make_sc_agg(with_gather):
  mesh = plsc.VectorSubcoreMesh(core_axis_name="c", subcore_axis_name="s",
                                num_cores=NC, num_subcores=NS)

  scratch = [
      pltpu.VMEM((CHUNKS, CH), jnp.int32),
      pltpu.VMEM((CHUNKS, CH), jnp.int32),
      pltpu.VMEM((NBUF, CH, DIM), jnp.float32),
      pltpu.VMEM_SHARED((NP, DIM), jnp.float32),
      pltpu.VMEM_SHARED((N, DIM), jnp.float32),
      pltpu.SemaphoreType.DMA((NBUF,)),
  ]

  def body(y_hbm, src_hbm, dst_hbm, zeros_hbm, ones_hbm, out_hbm,
           src_v, dst_v, rows_v, z_sh, y_sh, sem):
    c = lax.axis_index("c")
    s = lax.axis_index("s")
    wid = s * NC + c

    _over_rows(s, lambda r: pltpu.sync_copy(zeros_hbm.at[r], z_sh.at[r]))
    if with_gather:
      _over_rows(s, lambda r: pltpu.sync_copy(y_hbm.at[r], y_sh.at[r]))
    else:
      pltpu.sync_copy(ones_hbm, rows_v.at[0])

    pltpu.sync_copy(src_hbm.at[wid], src_v)
    pltpu.sync_copy(dst_hbm.at[wid], dst_v)
    plsc.subcore_barrier()

    if with_gather:
      for b in range(NBUF):
        pltpu.async_copy(y_sh.at[src_v.at[b]], rows_v.at[b], sem.at[b])

      def outer(o, _):
        for b in range(NBUF):
          j = o * NBUF + b
          pltpu.make_async_copy(
              y_sh.at[src_v.at[j]], rows_v.at[b], sem.at[b]).wait()
          pltpu.sync_copy(rows_v.at[b], z_sh.at[dst_v.at[j]], add=True)
          nxt = j + NBUF

          @pl.when(nxt < CHUNKS)
          def _():
            pltpu.async_copy(y_sh.at[src_v.at[nxt]], rows_v.at[b],
                             sem.at[b])
        return 0

      lax.fori_loop(0, NOUT, outer, 0)
    else:

      def step(j, _):
        pltpu.sync_copy(rows_v.at[0], z_sh.at[dst_v.at[j]], add=True)
        return 0

      lax.fori_loop(0, CHUNKS, step, 0)
    plsc.subcore_barrier()
    _over_rows(s, lambda r: pltpu.sync_copy(z_sh.at[r], out_hbm.at[c, r]))

  return pl.kernel(
      body,
      out_type=jax.ShapeDtypeStruct((NC, N, DIM), jnp.float32),
      mesh=mesh,
      scratch_types=scratch,
      compiler_params=pltpu.CompilerParams(use_tc_tiling_on_sc=False),
  )


BT = 2000
GRID = N // BT


def _tc_prep1_body(d_ref, x_ref, w_ref, y_ref, dinv_ref):
  deg = d_ref[0] + d_ref[1] + 1.0
  dinv = lax.rsqrt(deg)
  dinv_ref[...] = dinv
  y_ref[...] = dinv * jnp.dot(x_ref[...], w_ref[...],
                              preferred_element_type=jnp.float32)


_tc_prep1 = pl.pallas_call(
    _tc_prep1_body,
    grid=(GRID,),
    in_specs=[
        pl.BlockSpec((NC, BT, DIM), lambda i: (0, i, 0)),
        pl.BlockSpec((BT, D_IN), lambda i: (i, 0)),
        pl.BlockSpec((D_IN, DIM), lambda i: (0, 0)),
    ],
    out_specs=[
        pl.BlockSpec((BT, DIM), lambda i: (i, 0)),
        pl.BlockSpec((BT, DIM), lambda i: (i, 0)),
    ],
    out_shape=[
        jax.ShapeDtypeStruct((N, DIM), jnp.float32),
        jax.ShapeDtypeStruct((N, DIM), jnp.float32),
    ],
)


def _tc_mid_body(z_ref, yin_ref, dinv_ref, b_ref, w_ref, y_ref):
  dinv = dinv_ref[...]
  h = jax.nn.relu(dinv * (z_ref[0] + z_ref[1] + yin_ref[...]) + b_ref[...])
  y_ref[...] = dinv * jnp.dot(h, w_ref[...],
                              preferred_element_type=jnp.float32)


_tc_mid = pl.pallas_call(
    _tc_mid_body,
    grid=(GRID,),
    in_specs=[
        pl.BlockSpec((NC, BT, DIM), lambda i: (0, i, 0)),
        pl.BlockSpec((BT, DIM), lambda i: (i, 0)),
        pl.BlockSpec((BT, DIM), lambda i: (i, 0)),
        pl.BlockSpec((1, DIM), lambda i: (0, 0)),
        pl.BlockSpec((DIM, DIM), lambda i: (0, 0)),
    ],
    out_specs=pl.BlockSpec((BT, DIM), lambda i: (i, 0)),
    out_shape=jax.ShapeDtypeStruct((N, DIM), jnp.float32),
)


def _tc_final_body(z_ref, yin_ref, dinv_ref, b_ref, wl_ref, bl_ref, o_ref):
  h = dinv_ref[...] * (z_ref[0] + z_ref[1] + yin_ref[...]) + b_ref[...]
  lg = jnp.dot(h, wl_ref[...], preferred_element_type=jnp.float32)
  lg = lg + bl_ref[...]
  m = jnp.max(lg, axis=1, keepdims=True)
  lse = m + jnp.log(jnp.sum(jnp.exp(lg - m), axis=1, keepdims=True))
  o_ref[...] = lg - lse


_tc_final = pl.pallas_call(
    _tc_final_body,
    grid=(GRID,),
    in_specs=[
        pl.BlockSpec((NC, BT, DIM), lambda i: (0, i, 0)),
        pl.BlockSpec((BT, DIM), lambda i: (i, 0)),
        pl.BlockSpec((BT, DIM), lambda i: (i, 0)),
        pl.BlockSpec((1, DIM), lambda i: (0, 0)),
        pl.BlockSpec((DIM, N_CLASSES), lambda i: (0, 0)),
        pl.BlockSpec((1, N_CLASSES), lambda i: (0, 0)),
    ],
    out_specs=pl.BlockSpec((BT, N_CLASSES), lambda i: (i, 0)),
    out_shape=jax.ShapeDtypeStruct((N, N_CLASSES), jnp.float32),
)


def kernel(x, edge_index, W1, b1, W2, b2, W3, b3, Wl, bl):
  padt = (E_PAD - E) // NT
  src_t = edge_index[0].reshape(NT, E // NT)
  dst_t = edge_index[1].reshape(NT, E // NT)
  pad_src = jnp.zeros((NT, padt), jnp.int32)
  pad_dst = jnp.broadcast_to(
      N + (jnp.arange(padt, dtype=jnp.int32) % SCRAP), (NT, padt))
  src_p = jnp.concatenate([src_t, pad_src], axis=1).reshape(NT, CHUNKS, CH)
  dst_p = jnp.concatenate([dst_t, pad_dst], axis=1).reshape(NT, CHUNKS, CH)

  zeros = jnp.zeros((N, DIM), jnp.float32)
  ones = jnp.ones((CH, DIM), jnp.float32)
  dummy_y = zeros

  sc_layer = _make_sc_agg(True)
  sc_deg = _make_sc_agg(False)

  deg_parts = sc_deg(dummy_y, src_p, dst_p, zeros, ones)
  y1, dinv = _tc_prep1(deg_parts, x, W1)
  z1 = sc_layer(y1, src_p, dst_p, zeros, ones)
  y2 = _tc_mid(z1, y1, dinv, b1.reshape(1, DIM), W2)
  z2 = sc_layer(y2, src_p, dst_p, zeros, ones)
  y3 = _tc_mid(z2, y2, dinv, b2.reshape(1, DIM), W3)
  z3 = sc_layer(y3, src_p, dst_p, zeros, ones)
  return _tc_final(z3, y3, dinv, b3.reshape(1, DIM), Wl,
                   bl.reshape(1, N_CLASSES))


# speedup vs baseline: 71.8351x; 1.3575x over previous
import functools

import jax
import jax.numpy as jnp
from jax import lax
from jax.experimental import pallas as pl
from jax.experimental.pallas import tpu as pltpu
from jax.experimental.pallas import tpu_sc as plsc

N = 10000
D_IN = 128
DIM = 16
N_CLASSES = 16
E = 320000

NC = 2
NS = 16
NT = NC * NS
CH = 128
CHUNKS = 80
NBUF = 8
NOUT = CHUNKS // NBUF
E_PAD = NT * CHUNKS * CH
SCRAP = 512
NP = N + SCRAP
RPT = 632
RPT_LAST = N - (NS - 1) * RPT


def _over_rows(s, fn):
  @pl.when(s < NS - 1)
  def _():
    fn(pl.ds(s * RPT, RPT))

  @pl.when(s == NS - 1)
  def _():
    fn(pl.ds((NS - 1) * RPT, RPT_LAST))

@functools.lru_cache(maxsize=None)
def _make_sc_agg(with_gather):
  mesh = plsc.VectorSubcoreMesh(core_axis_name="c", subcore_axis_name="s",
                                num_cores=NC, num_subcores=NS)

  scratch = [
      pltpu.VMEM((CHUNKS, CH), jnp.int32),
      pltpu.VMEM((CHUNKS, CH), jnp.int32),
      pltpu.VMEM((NBUF, CH, DIM), jnp.float32),
      pltpu.VMEM_SHARED((NP, DIM), jnp.float32),
      pltpu.VMEM_SHARED((N, DIM), jnp.float32),
      pltpu.SemaphoreType.DMA((NBUF,)),
  ]

  def body(y_hbm, src_hbm, dst_hbm, zeros_hbm, ones_hbm, out_hbm,
           src_v, dst_v, rows_v, z_sh, y_sh, sem):
    c = lax.axis_index("c")
    s = lax.axis_index("s")
    wid = s * NC + c

    _over_rows(s, lambda r: pltpu.sync_copy(zeros_hbm.at[r], z_sh.at[r]))
    if with_gather:
      _over_rows(s, lambda r: pltpu.sync_copy(y_hbm.at[r], y_sh.at[r]))
    else:
      pltpu.sync_copy(ones_hbm, rows_v.at[0])

    pltpu.sync_copy(src_hbm.at[wid], src_v)
    pltpu.sync_copy(dst_hbm.at[wid], dst_v)
    plsc.subcore_barrier()

    if with_gather:
      for b in range(NBUF):
        pltpu.async_copy(y_sh.at[src_v.at[b]], rows_v.at[b], sem.at[b])

      def outer(o, _):
        for b in range(NBUF):
          j = o * NBUF + b
          pltpu.make_async_copy(
              y_sh.at[src_v.at[j]], rows_v.at[b], sem.at[b]).wait()
          pltpu.sync_copy(rows_v.at[b], z_sh.at[dst_v.at[j]], add=True)
          nxt = j + NBUF

          @pl.when(nxt < CHUNKS)
          def _():
            pltpu.async_copy(y_sh.at[src_v.at[nxt]], rows_v.at[b],
                             sem.at[b])
        return 0

      lax.fori_loop(0, NOUT, outer, 0)
    else:

      def step(j, _):
        pltpu.sync_copy(rows_v.at[0], z_sh.at[dst_v.at[j]], add=True)
        return 0

      lax.fori_loop(0, CHUNKS, step, 0)
    plsc.subcore_barrier()
    _over_rows(s, lambda r: pltpu.sync_copy(z_sh.at[r], out_hbm.at[c, r]))

  return pl.kernel(
      body,
      out_type=jax.ShapeDtypeStruct((NC, N, DIM), jnp.float32),
      mesh=mesh,
      scratch_types=scratch,
      compiler_params=pltpu.CompilerParams(use_tc_tiling_on_sc=False),
  )


P = 8
NB = N // P
LANES = P * DIM


def _blockdiag(w):
  k, m = w.shape
  return (jnp.eye(P, dtype=w.dtype)[:, None, :, None]
          * w[None, :, None, :]).reshape(P * k, P * m)


def _tc_prep1_body(d_ref, x8_ref, w_ref, y_ref, dinv_ref):
  deg = d_ref[0] + d_ref[1] + 1.0
  dinv = lax.rsqrt(deg)
  dinv_ref[...] = dinv
  y_ref[...] = dinv * jnp.dot(x8_ref[...], w_ref[...],
                              preferred_element_type=jnp.float32)


_tc_prep1 = pl.pallas_call(
    _tc_prep1_body,
    out_shape=[
        jax.ShapeDtypeStruct((NB, LANES), jnp.float32),
        jax.ShapeDtypeStruct((NB, LANES), jnp.float32),
    ],
)


def _tc_mid_body(z_ref, yin_ref, dinv_ref, b_ref, w_ref, y_ref):
  dinv = dinv_ref[...]
  h = jax.nn.relu(dinv * (z_ref[0] + z_ref[1] + yin_ref[...]) + b_ref[...])
  y_ref[...] = dinv * jnp.dot(h, w_ref[...],
                              preferred_element_type=jnp.float32)


_tc_mid = pl.pallas_call(
    _tc_mid_body,
    out_shape=jax.ShapeDtypeStruct((NB, LANES), jnp.float32),
)


def _tc_final_body(z_ref, yin_ref, dinv_ref, b_ref, wl_ref, bl_ref, g_ref,
                   o_ref):
  h = dinv_ref[...] * (z_ref[0] + z_ref[1] + yin_ref[...]) + b_ref[...]
  lg = jnp.dot(h, wl_ref[...], preferred_element_type=jnp.float32)
  lg = lg + bl_ref[...]
  m = jnp.max(lg, axis=1, keepdims=True)
  ex = jnp.exp(lg - m)
  s = jnp.dot(ex, g_ref[...], preferred_element_type=jnp.float32)
  o_ref[...] = (lg - m) - jnp.log(s)


_tc_final = pl.pallas_call(
    _tc_final_body,
    out_shape=jax.ShapeDtypeStruct((NB, LANES), jnp.float32),
)


def kernel(x, edge_index, W1, b1, W2, b2, W3, b3, Wl, bl):
  padt = (E_PAD - E) // NT
  src_t = edge_index[0].reshape(NT, E // NT)
  dst_t = edge_index[1].reshape(NT, E // NT)
  pad_src = jnp.zeros((NT, padt), jnp.int32)
  pad_dst = jnp.broadcast_to(
      N + (jnp.arange(padt, dtype=jnp.int32) % SCRAP), (NT, padt))
  src_p = jnp.concatenate([src_t, pad_src], axis=1).reshape(NT, CHUNKS, CH)
  dst_p = jnp.concatenate([dst_t, pad_dst], axis=1).reshape(NT, CHUNKS, CH)

  zeros = jnp.zeros((N, DIM), jnp.float32)
  ones = jnp.ones((CH, DIM), jnp.float32)
  dummy_y = zeros

  sc_layer = _make_sc_agg(True)
  sc_deg = _make_sc_agg(False)

  x8 = x.reshape(NB, P * D_IN)
  w1b = _blockdiag(W1)
  w2b = _blockdiag(W2)
  w3b = _blockdiag(W3)
  wlb = _blockdiag(Wl)
  gmat = _blockdiag(jnp.ones((DIM, N_CLASSES), jnp.float32))
  b1p = jnp.tile(b1, P).reshape(1, LANES)
  b2p = jnp.tile(b2, P).reshape(1, LANES)
  b3p = jnp.tile(b3, P).reshape(1, LANES)
  blp = jnp.tile(bl, P).reshape(1, LANES)

  deg_parts = sc_deg(dummy_y, src_p, dst_p, zeros, ones)
  y1, dinv = _tc_prep1(deg_parts.reshape(NC, NB, LANES), x8, w1b)
  z1 = sc_layer(y1.reshape(N, DIM), src_p, dst_p, zeros, ones)
  y2 = _tc_mid(z1.reshape(NC, NB, LANES), y1, dinv, b1p, w2b)
  z2 = sc_layer(y2.reshape(N, DIM), src_p, dst_p, zeros, ones)
  y3 = _tc_mid(z2.reshape(NC, NB, LANES), y2, dinv, b2p, w3b)
  z3 = sc_layer(y3.reshape(N, DIM), src_p, dst_p, zeros, ones)
  out = _tc_final(z3.reshape(NC, NB, LANES), y3, dinv, b3p, wlb, blp, gmat)
  return out.reshape(N, N_CLASSES)


# speedup vs baseline: 77.9480x; 1.0851x over previous
import functools

import jax
import jax.numpy as jnp
from jax import lax
from jax.experimental import pallas as pl
from jax.experimental.pallas import tpu as pltpu
from jax.experimental.pallas import tpu_sc as plsc

N = 10000
D_IN = 128
DIM = 16
N_CLASSES = 16
E = 320000

NC = 2
NS = 16
NT = NC * NS
CH = 128
CHUNKS = 80
NBUF = 8
NOUT = CHUNKS // NBUF
E_PAD = NT * CHUNKS * CH
SCRAP = 512
NP = N + SCRAP
RPT = 632
RPT_LAST = N - (NS - 1) * RPT


def _over_rows(s, fn):
  @pl.when(s < NS - 1)
  def _():
    fn(pl.ds(s * RPT, RPT))

  @pl.when(s == NS - 1)
  def _():
    fn(pl.ds((NS - 1) * RPT, RPT_LAST))

@functools.lru_cache(maxsize=None)
def _make_sc_agg(with_gather):
  mesh = plsc.VectorSubcoreMesh(core_axis_name="c", subcore_axis_name="s",
                                num_cores=NC, num_subcores=NS)

  scratch = [
      pltpu.VMEM((CHUNKS, CH), jnp.int32),
      pltpu.VMEM((CHUNKS, CH), jnp.int32),
      pltpu.VMEM((NBUF, CH, DIM), jnp.float32),
      pltpu.VMEM_SHARED((NP, DIM), jnp.float32),
      pltpu.VMEM_SHARED((N, DIM), jnp.float32),
      pltpu.SemaphoreType.DMA((NBUF,)),
  ]

  def body(y_hbm, src_hbm, dst_hbm, zeros_hbm, ones_hbm, out0_hbm, out1_hbm,
           src_v, dst_v, rows_v, z_sh, y_sh, sem):
    c = lax.axis_index("c")
    s = lax.axis_index("s")
    wid = s * NC + c

    _over_rows(s, lambda r: pltpu.sync_copy(zeros_hbm.at[r], z_sh.at[r]))
    if with_gather:
      _over_rows(s, lambda r: pltpu.sync_copy(y_hbm.at[r], y_sh.at[r]))
    else:
      pltpu.sync_copy(ones_hbm, rows_v.at[0])

    pltpu.sync_copy(src_hbm.at[wid], src_v)
    pltpu.sync_copy(dst_hbm.at[wid], dst_v)
    plsc.subcore_barrier()

    if with_gather:
      def start_gather(b, j):
        from_hbm = (j % 5) < 2

        @pl.when(from_hbm)
        def _():
          pltpu.async_copy(y_hbm.at[src_v.at[j]], rows_v.at[b], sem.at[b])

        @pl.when(jnp.logical_not(from_hbm))
        def _():
          pltpu.async_copy(y_sh.at[src_v.at[j]], rows_v.at[b], sem.at[b])

      for b in range(NBUF):
        start_gather(b, b)

      def outer(o, _):
        for b in range(NBUF):
          j = o * NBUF + b
          pltpu.make_async_copy(
              y_sh.at[src_v.at[j]], rows_v.at[b], sem.at[b]).wait()
          pltpu.sync_copy(rows_v.at[b], z_sh.at[dst_v.at[j]], add=True)
          nxt = j + NBUF

          @pl.when(nxt < CHUNKS)
          def _():
            start_gather(b, nxt)
        return 0

      lax.fori_loop(0, NOUT, outer, 0)
    else:

      def step(j, _):
        pltpu.sync_copy(rows_v.at[0], z_sh.at[dst_v.at[j]], add=True)
        return 0

      lax.fori_loop(0, CHUNKS, step, 0)
    plsc.subcore_barrier()

    @pl.when(c == 0)
    def _():
      _over_rows(s, lambda r: pltpu.sync_copy(z_sh.at[r], out0_hbm.at[r]))

    @pl.when(c != 0)
    def _():
      _over_rows(s, lambda r: pltpu.sync_copy(z_sh.at[r], out1_hbm.at[r]))

  return pl.kernel(
      body,
      out_type=[
          jax.ShapeDtypeStruct((N, DIM), jnp.float32),
          jax.ShapeDtypeStruct((N, DIM), jnp.float32),
      ],
      mesh=mesh,
      scratch_types=scratch,
      compiler_params=pltpu.CompilerParams(use_tc_tiling_on_sc=False),
  )


P = 8
NB = N // P
LANES = P * DIM


def _blockdiag(w):
  k, m = w.shape
  return (jnp.eye(P, dtype=w.dtype)[:, None, :, None]
          * w[None, :, None, :]).reshape(P * k, P * m)


def _tc_prep1_body(d0_ref, d1_ref, x8_ref, w_ref, y_ref, dinv_ref):
  deg = d0_ref[...] + d1_ref[...] + 1.0
  dinv = lax.rsqrt(deg)
  dinv_ref[...] = dinv
  y_ref[...] = dinv * jnp.dot(x8_ref[...], w_ref[...],
                              preferred_element_type=jnp.float32)


_tc_prep1 = pl.pallas_call(
    _tc_prep1_body,
    out_shape=[
        jax.ShapeDtypeStruct((NB, LANES), jnp.float32),
        jax.ShapeDtypeStruct((NB, LANES), jnp.float32),
    ],
)


def _tc_mid_body(z0_ref, z1_ref, yin_ref, dinv_ref, b_ref, w_ref, y_ref):
  dinv = dinv_ref[...]
  h = jax.nn.relu(dinv * (z0_ref[...] + z1_ref[...] + yin_ref[...])
                  + b_ref[...])
  y_ref[...] = dinv * jnp.dot(h, w_ref[...],
                              preferred_element_type=jnp.float32)


_tc_mid = pl.pallas_call(
    _tc_mid_body,
    out_shape=jax.ShapeDtypeStruct((NB, LANES), jnp.float32),
)


def _tc_final_body(z0_ref, z1_ref, yin_ref, dinv_ref, b_ref, wl_ref, bl_ref,
                   g_ref, o_ref):
  h = dinv_ref[...] * (z0_ref[...] + z1_ref[...] + yin_ref[...]) + b_ref[...]
  lg = jnp.dot(h, wl_ref[...], preferred_element_type=jnp.float32)
  lg = lg + bl_ref[...]
  m = jnp.max(lg, axis=1, keepdims=True)
  ex = jnp.exp(lg - m)
  s = jnp.dot(ex, g_ref[...], preferred_element_type=jnp.float32)
  o_ref[...] = (lg - m) - jnp.log(s)


_tc_final = pl.pallas_call(
    _tc_final_body,
    out_shape=jax.ShapeDtypeStruct((NB, LANES), jnp.float32),
)


def kernel(x, edge_index, W1, b1, W2, b2, W3, b3, Wl, bl):
  padt = (E_PAD - E) // NT
  src_t = edge_index[0].reshape(NT, E // NT)
  dst_t = edge_index[1].reshape(NT, E // NT)
  pad_src = jnp.zeros((NT, padt), jnp.int32)
  pad_dst = jnp.broadcast_to(
      N + (jnp.arange(padt, dtype=jnp.int32) % SCRAP), (NT, padt))
  src_p = jnp.concatenate([src_t, pad_src], axis=1).reshape(NT, CHUNKS, CH)
  dst_p = jnp.concatenate([dst_t, pad_dst], axis=1).reshape(NT, CHUNKS, CH)

  zeros = jnp.zeros((N, DIM), jnp.float32)
  ones = jnp.ones((CH, DIM), jnp.float32)
  dummy_y = zeros

  sc_layer = _make_sc_agg(True)
  sc_deg = _make_sc_agg(False)

  x8 = x.reshape(NB, P * D_IN)
  w1b = _blockdiag(W1)
  w2b = _blockdiag(W2)
  w3b = _blockdiag(W3)
  wlb = _blockdiag(Wl)
  gmat = _blockdiag(jnp.ones((DIM, N_CLASSES), jnp.float32))
  b1p = jnp.tile(b1, P).reshape(1, LANES)
  b2p = jnp.tile(b2, P).reshape(1, LANES)
  b3p = jnp.tile(b3, P).reshape(1, LANES)
  blp = jnp.tile(bl, P).reshape(1, LANES)

  pk = lambda a: a.reshape(NB, LANES)

  d0, d1 = sc_deg(dummy_y, src_p, dst_p, zeros, ones)
  y1, dinv = _tc_prep1(pk(d0), pk(d1), x8, w1b)
  z0, z1 = sc_layer(y1.reshape(N, DIM), src_p, dst_p, zeros, ones)
  y2 = _tc_mid(pk(z0), pk(z1), y1, dinv, b1p, w2b)
  z0, z1 = sc_layer(y2.reshape(N, DIM), src_p, dst_p, zeros, ones)
  y3 = _tc_mid(pk(z0), pk(z1), y2, dinv, b2p, w3b)
  z0, z1 = sc_layer(y3.reshape(N, DIM), src_p, dst_p, zeros, ones)
  out = _tc_final(pk(z0), pk(z1), y3, dinv, b3p, wlb, blp, gmat)
  return out.reshape(N, N_CLASSES)


# speedup vs baseline: 80.6752x; 1.0350x over previous
import functools

import jax
import jax.numpy as jnp
from jax import lax
from jax.experimental import pallas as pl
from jax.experimental.pallas import tpu as pltpu
from jax.experimental.pallas import tpu_sc as plsc

N = 10000
D_IN = 128
DIM = 16
N_CLASSES = 16
E = 320000

NC = 2
NS = 16
NT = NC * NS
CH = 125
CHUNKS = 80
NBUF = 8
NOUT = CHUNKS // NBUF
NP = N
RPT = 632
RPT_LAST = N - (NS - 1) * RPT


def _over_rows(s, fn):
  @pl.when(s < NS - 1)
  def _():
    fn(pl.ds(s * RPT, RPT))

  @pl.when(s == NS - 1)
  def _():
    fn(pl.ds((NS - 1) * RPT, RPT_LAST))

@functools.lru_cache(maxsize=None)
def _make_sc_agg(with_gather):
  mesh = plsc.VectorSubcoreMesh(core_axis_name="c", subcore_axis_name="s",
                                num_cores=NC, num_subcores=NS)

  scratch = [
      pltpu.VMEM((CHUNKS, CH), jnp.int32),
      pltpu.VMEM((CHUNKS, CH), jnp.int32),
      pltpu.VMEM((NBUF, CH, DIM), jnp.float32),
      pltpu.VMEM_SHARED((NP, DIM), jnp.float32),
      pltpu.VMEM_SHARED((N, DIM), jnp.float32),
      pltpu.SemaphoreType.DMA((NBUF,)),
  ]

  def body(y_hbm, src_hbm, dst_hbm, zeros_hbm, ones_hbm, out0_hbm, out1_hbm,
           src_v, dst_v, rows_v, z_sh, y_sh, sem):
    c = lax.axis_index("c")
    s = lax.axis_index("s")
    wid = s * NC + c

    _over_rows(s, lambda r: pltpu.sync_copy(zeros_hbm.at[r], z_sh.at[r]))
    if with_gather:
      _over_rows(s, lambda r: pltpu.sync_copy(y_hbm.at[r], y_sh.at[r]))
    else:
      pltpu.sync_copy(ones_hbm, rows_v.at[0])

    pltpu.sync_copy(src_hbm.at[wid], src_v)
    pltpu.sync_copy(dst_hbm.at[wid], dst_v)
    plsc.subcore_barrier()

    if with_gather:
      def start_gather(b, j):
        from_hbm = (j % 5) < 2

        @pl.when(from_hbm)
        def _():
          pltpu.async_copy(y_hbm.at[src_v.at[j]], rows_v.at[b], sem.at[b])

        @pl.when(jnp.logical_not(from_hbm))
        def _():
          pltpu.async_copy(y_sh.at[src_v.at[j]], rows_v.at[b], sem.at[b])

      for b in range(NBUF):
        start_gather(b, b)

      def outer(o, _):
        for b in range(NBUF):
          j = o * NBUF + b
          pltpu.make_async_copy(
              y_sh.at[src_v.at[j]], rows_v.at[b], sem.at[b]).wait()
          pltpu.sync_copy(rows_v.at[b], z_sh.at[dst_v.at[j]], add=True)
          nxt = j + NBUF

          @pl.when(nxt < CHUNKS)
          def _():
            start_gather(b, nxt)
        return 0

      lax.fori_loop(0, NOUT, outer, 0)
    else:

      def step(j, _):
        pltpu.sync_copy(rows_v.at[0], z_sh.at[dst_v.at[j]], add=True)
        return 0

      lax.fori_loop(0, CHUNKS, step, 0)
    plsc.subcore_barrier()

    @pl.when(c == 0)
    def _():
      _over_rows(s, lambda r: pltpu.sync_copy(z_sh.at[r], out0_hbm.at[r]))

    @pl.when(c != 0)
    def _():
      _over_rows(s, lambda r: pltpu.sync_copy(z_sh.at[r], out1_hbm.at[r]))

  return pl.kernel(
      body,
      out_type=[
          jax.ShapeDtypeStruct((N, DIM), jnp.float32),
          jax.ShapeDtypeStruct((N, DIM), jnp.float32),
      ],
      mesh=mesh,
      scratch_types=scratch,
      compiler_params=pltpu.CompilerParams(use_tc_tiling_on_sc=False),
  )


P = 8
NB = N // P
LANES = P * DIM


def _blockdiag(w):
  k, m = w.shape
  return (jnp.eye(P, dtype=w.dtype)[:, None, :, None]
          * w[None, :, None, :]).reshape(P * k, P * m)


def _tc_mm1_body(x8_ref, w_ref, xt_ref):
  xt_ref[...] = jnp.dot(x8_ref[...], w_ref[...],
                        preferred_element_type=jnp.float32)


_tc_mm1 = pl.pallas_call(
    _tc_mm1_body,
    out_shape=jax.ShapeDtypeStruct((NB, LANES), jnp.float32),
)


def _tc_prep1_body(d0_ref, d1_ref, xt_ref, y_ref, dinv_ref):
  deg = d0_ref[...] + d1_ref[...] + 1.0
  dinv = lax.rsqrt(deg)
  dinv_ref[...] = dinv
  y_ref[...] = dinv * xt_ref[...]


_tc_prep1 = pl.pallas_call(
    _tc_prep1_body,
    out_shape=[
        jax.ShapeDtypeStruct((NB, LANES), jnp.float32),
        jax.ShapeDtypeStruct((NB, LANES), jnp.float32),
    ],
)


def _tc_mid_body(z0_ref, z1_ref, yin_ref, dinv_ref, b_ref, w_ref, y_ref):
  dinv = dinv_ref[...]
  h = jax.nn.relu(dinv * (z0_ref[...] + z1_ref[...] + yin_ref[...])
                  + b_ref[...])
  y_ref[...] = dinv * jnp.dot(h, w_ref[...],
                              preferred_element_type=jnp.float32)


_tc_mid = pl.pallas_call(
    _tc_mid_body,
    out_shape=jax.ShapeDtypeStruct((NB, LANES), jnp.float32),
)


def _tc_final_body(z0_ref, z1_ref, yin_ref, dinv_ref, b_ref, wl_ref, bl_ref,
                   g_ref, o_ref):
  h = dinv_ref[...] * (z0_ref[...] + z1_ref[...] + yin_ref[...]) + b_ref[...]
  lg = jnp.dot(h, wl_ref[...], preferred_element_type=jnp.float32)
  lg = lg + bl_ref[...]
  m = jnp.max(lg, axis=1, keepdims=True)
  ex = jnp.exp(lg - m)
  s = jnp.dot(ex, g_ref[...], preferred_element_type=jnp.float32)
  o_ref[...] = (lg - m) - jnp.log(s)


_tc_final = pl.pallas_call(
    _tc_final_body,
    out_shape=jax.ShapeDtypeStruct((NB, LANES), jnp.float32),
)


def kernel(x, edge_index, W1, b1, W2, b2, W3, b3, Wl, bl):
  src_p = edge_index[0].reshape(NT, CHUNKS, CH)
  dst_p = edge_index[1].reshape(NT, CHUNKS, CH)

  zeros = jnp.zeros((N, DIM), jnp.float32)
  ones = jnp.ones((CH, DIM), jnp.float32)
  dummy_y = zeros

  sc_layer = _make_sc_agg(True)
  sc_deg = _make_sc_agg(False)

  x8 = x.reshape(NB, P * D_IN)
  w1b = _blockdiag(W1)
  w2b = _blockdiag(W2)
  w3b = _blockdiag(W3)
  wlb = _blockdiag(Wl)
  gmat = _blockdiag(jnp.ones((DIM, N_CLASSES), jnp.float32))
  b1p = jnp.tile(b1, P).reshape(1, LANES)
  b2p = jnp.tile(b2, P).reshape(1, LANES)
  b3p = jnp.tile(b3, P).reshape(1, LANES)
  blp = jnp.tile(bl, P).reshape(1, LANES)

  pk = lambda a: a.reshape(NB, LANES)

  xt1 = _tc_mm1(x8, w1b)
  d0, d1 = sc_deg(dummy_y, src_p, dst_p, zeros, ones)
  y1, dinv = _tc_prep1(pk(d0), pk(d1), xt1)
  z0, z1 = sc_layer(y1.reshape(N, DIM), src_p, dst_p, zeros, ones)
  y2 = _tc_mid(pk(z0), pk(z1), y1, dinv, b1p, w2b)
  z0, z1 = sc_layer(y2.reshape(N, DIM), src_p, dst_p, zeros, ones)
  y3 = _tc_mid(pk(z0), pk(z1), y2, dinv, b2p, w3b)
  z0, z1 = sc_layer(y3.reshape(N, DIM), src_p, dst_p, zeros, ones)
  out = _tc_final(pk(z0), pk(z1), y3, dinv, b3p, wlb, blp, gmat)
  return out.reshape(N, N_CLASSES)
